# Initial kernel scaffold; baseline (speedup 1.0000x reference)
import jax
import jax.numpy as jnp
from jax.experimental import pallas as pl


def kernel(x, edge_index, W1, b1, W2, b2, W3, b3, bn1_g, bn1_b, bn2_g, bn2_b):
    raise NotImplementedError("write your pallas kernel here")


# speedup vs baseline: 42.5553x; 42.5553x over previous
import functools
import numpy as np
import jax
import jax.numpy as jnp
from jax import lax
from jax.experimental import pallas as pl
from jax.experimental.pallas import tpu as pltpu
from jax.experimental.pallas import tpu_sc as plsc

N_NODES = 10000
N_EDGES = 320000
FEAT = 128
SAMPLE = 4096
NW = 32
EPW = N_EDGES // NW
EPAD = 10240
QPW = SAMPLE // NW
CAP = EPW + 16
DUMPSLOT = EPW + 8
DUMP = SAMPLE
TROWS = SAMPLE + 8
STRIPE = 264

F32 = jnp.float32
I32 = jnp.int32

MESH = dict(mesh=plsc.VectorSubcoreMesh(core_axis_name="c", subcore_axis_name="s",
                                        num_cores=2, num_subcores=16))


def _ids():
    cid = lax.axis_index("c")
    sid = lax.axis_index("s")
    return cid, sid, cid * 16 + sid


def _iota16():
    return lax.iota(I32, 16)


@functools.partial(
    pl.kernel, out_type=jax.ShapeDtypeStruct((2 * N_NODES,), F32), **MESH,
    scratch_types=[pltpu.VMEM_SHARED((N_NODES,), F32),
                   pltpu.VMEM((EPW,), I32), pltpu.VMEM((EPW,), F32),
                   pltpu.VMEM((EPW,), F32), pltpu.SemaphoreType.DMA])
def _k1_deg(row_hbm, out_hbm, hist_sp, ebuf, ones_v, zbuf, sem):
    cid, sid, wid = _ids()
    pltpu.sync_copy(row_hbm.at[pl.ds(wid * EPW, EPW)], ebuf)

    def fill(i, c):
        ones_v[pl.ds(i * 16, 16)] = jnp.full((16,), 1.0, F32)
        zbuf[pl.ds(i * 16, 16)] = jnp.full((16,), 0.0, F32)
        return c
    lax.fori_loop(0, EPW // 16, fill, jnp.int32(0))

    @pl.when(sid == 0)
    def _():
        pltpu.sync_copy(zbuf, hist_sp)
    plsc.subcore_barrier()
    pltpu.async_copy(ones_v, hist_sp.at[ebuf], sem, add=True).wait()
    plsc.subcore_barrier()

    @pl.when(sid == 0)
    def _():
        pltpu.sync_copy(hist_sp, zbuf)
        pltpu.sync_copy(zbuf, out_hbm.at[pl.ds(cid * N_NODES, N_NODES)])


@functools.partial(
    pl.kernel, out_type=jax.ShapeDtypeStruct((SAMPLE,), I32), **MESH,
    scratch_types=[pltpu.VMEM((QPW,), F32), pltpu.VMEM((QPW,), I32),
                   pltpu.VMEM((QPW,), I32)]
    + [pltpu.VMEM((16,), F32) for _ in range(8)]
    + [pltpu.SemaphoreType.DMA for _ in range(8)])
def _k2_search(pc_hbm, r_hbm, out_hbm, rbuf, lobuf, nbuf,
               g0, g1, g2, g3, g4, g5, g6, g7,
               s0, s1, s2, s3, s4, s5, s6, s7):
    cid, sid, wid = _ids()
    gbs = (g0, g1, g2, g3, g4, g5, g6, g7)
    sems = (s0, s1, s2, s3, s4, s5, s6, s7)
    pltpu.sync_copy(r_hbm.at[pl.ds(wid * QPW, QPW)], rbuf)
    for g in range(8):
        lobuf[pl.ds(16 * g, 16)] = jnp.full((16,), 0, I32)
        nbuf[pl.ds(16 * g, 16)] = jnp.full((16,), N_NODES, I32)
    for _step in range(14):
        descs = []
        for g in range(8):
            lo = lobuf[pl.ds(16 * g, 16)]
            n = nbuf[pl.ds(16 * g, 16)]
            mid = lo + (n >> 1)
            midc = jnp.minimum(mid, N_NODES - 1)
            descs.append(pltpu.async_copy(pc_hbm.at[midc], gbs[g], sems[g]))
        for g in range(8):
            descs[g].wait()
            lo = lobuf[pl.ds(16 * g, 16)]
            n = nbuf[pl.ds(16 * g, 16)]
            r = rbuf[pl.ds(16 * g, 16)]
            half = n >> 1
            mid = lo + half
            v = gbs[g][...]
            lt = v < r
            pos = n > 0
            lobuf[pl.ds(16 * g, 16)] = jnp.where(lt & pos, mid + 1, lo)
            nbuf[pl.ds(16 * g, 16)] = jnp.where(pos, jnp.where(lt, n - half - 1, half), n)
    pltpu.sync_copy(lobuf, out_hbm.at[pl.ds(wid * QPW, QPW)])


@functools.partial(
    pl.kernel, out_type=jax.ShapeDtypeStruct((2 * N_NODES,), I32), **MESH,
    scratch_types=[pltpu.VMEM_SHARED((16 * N_NODES,), I32),
                   pltpu.VMEM((QPW,), I32), pltpu.VMEM((2 * QPW,), I32),
                   pltpu.VMEM((16,), I32), pltpu.VMEM((16,), I32),
                   pltpu.VMEM((16,), I32), pltpu.VMEM((N_NODES,), I32),
                   pltpu.VMEM((640,), I32), pltpu.VMEM((640,), I32),
                   pltpu.SemaphoreType.DMA])
def _k3_idxmap(s_hbm, out_hbm, flat_sp, sbuf, dbuf, mpbuf, repbuf, avbuf,
               negbuf, tb, accb, sem):
    cid, sid, wid = _ids()
    pltpu.sync_copy(s_hbm.at[pl.ds(wid * QPW, QPW)], sbuf)

    def nfill(i, c):
        negbuf[pl.ds(i * 16, 16)] = jnp.full((16,), -1, I32)
        return c
    lax.fori_loop(0, N_NODES // 16, nfill, jnp.int32(0))
    pltpu.sync_copy(negbuf, flat_sp.at[pl.ds(sid * N_NODES, N_NODES)])

    for j in range(8):
        v = sbuf[pl.ds(16 * j, 16)]
        dbuf[pl.ds(16 * j, 16)] = v
        dbuf[pl.ds(QPW + 16 * j, 16)] = v
    base = wid * QPW
    for c in range(8):
        v = sbuf[pl.ds(16 * c, 16)]
        pos = base + 16 * c + _iota16()
        mpbuf[...] = pos
        repbuf[...] = jnp.full((16,), 1, I32)

        def inner(k, carry):
            w = dbuf[pl.ds(16 * c + k, 16)]
            j2 = (16 * c + _iota16() + k) & (QPW - 1)
            rp = base + j2
            eq = w == v
            mp = mpbuf[...]
            mpbuf[...] = jnp.where(eq & (rp > mp), rp, mp)
            repbuf[...] = jnp.where(eq & (rp > pos), 0, repbuf[...])
            return carry
        lax.fori_loop(1, QPW, inner, jnp.int32(0))
        avbuf[...] = jnp.where(repbuf[...] == 1, mpbuf[...] + 1, 0)
        idx = v + sid * N_NODES
        pltpu.async_copy(avbuf, flat_sp.at[idx], sem, add=True).wait()
    plsc.subcore_barrier()

    off = jnp.minimum(640 * sid, N_NODES - 640)

    def mfill(i, c):
        accb[pl.ds(i * 16, 16)] = jnp.full((16,), -1, I32)
        return c
    lax.fori_loop(0, 40, mfill, jnp.int32(0))

    def merge(s, c):
        pltpu.sync_copy(flat_sp.at[pl.ds(s * N_NODES + off, 640)], tb)

        def mx(j, c2):
            accb[pl.ds(j * 16, 16)] = jnp.maximum(accb[pl.ds(j * 16, 16)],
                                                  tb[pl.ds(j * 16, 16)])
            return c2
        lax.fori_loop(0, 40, mx, jnp.int32(0))
        return c
    lax.fori_loop(0, 16, merge, jnp.int32(0))
    pltpu.sync_copy(accb, out_hbm.at[pl.ds(cid * N_NODES + off, 640)])


@functools.partial(
    pl.kernel,
    out_type=(jax.ShapeDtypeStruct((NW, EPAD), I32),
              jax.ShapeDtypeStruct((NW, EPAD), I32),
              jax.ShapeDtypeStruct((SAMPLE, FEAT), F32)), **MESH,
    scratch_types=[pltpu.VMEM((EPW,), I32), pltpu.VMEM((EPAD,), I32),
                   pltpu.VMEM((QPW,), I32), pltpu.VMEM((QPW, FEAT), F32),
                   pltpu.SemaphoreType.DMA, pltpu.SemaphoreType.DMA])
def _k4a_map(row_hbm, col_hbm, map_hbm, s_hbm, x_hbm,
             m0_hbm, m1_hbm, xs_hbm, ebuf, mbuf, sbuf, xsv, sem1, sem2):
    cid, sid, wid = _ids()
    for t in range(15):
        mbuf[pl.ds(EPW + 16 * t, 16)] = jnp.full((16,), -1, I32)
    pltpu.sync_copy(row_hbm.at[pl.ds(wid * EPW, EPW)], ebuf)
    pltpu.async_copy(map_hbm.at[ebuf], mbuf.at[pl.ds(0, EPW)], sem1).wait()
    pltpu.sync_copy(mbuf, m0_hbm.at[wid])
    pltpu.sync_copy(col_hbm.at[pl.ds(wid * EPW, EPW)], ebuf)
    pltpu.async_copy(map_hbm.at[ebuf], mbuf.at[pl.ds(0, EPW)], sem1).wait()
    pltpu.sync_copy(mbuf, m1_hbm.at[wid])
    pltpu.sync_copy(s_hbm.at[pl.ds(wid * QPW, QPW)], sbuf)
    pltpu.async_copy(x_hbm.at[sbuf], xsv, sem2).wait()
    pltpu.sync_copy(xsv, xs_hbm.at[pl.ds(wid * QPW, QPW)])


def _k4b_body(m0_ref, m1_ref, pos_ref, cnt_ref):
    i = pl.program_id(0)
    sid = lax.rem(i, 16)
    valid = (m0_ref[...] >= 0) & (m1_ref[...] >= 0)
    v2 = valid.astype(F32)
    lt128 = (lax.broadcasted_iota(I32, (128, 128), 0)
             < lax.broadcasted_iota(I32, (128, 128), 1)).astype(F32)
    inner = jnp.dot(v2, lt128, preferred_element_type=F32,
                    precision=lax.Precision.HIGHEST)
    rowtot = jnp.sum(v2, axis=1, keepdims=True)
    tri79 = (lax.broadcasted_iota(I32, (80, 80), 0)
             > lax.broadcasted_iota(I32, (80, 80), 1)).astype(F32)
    rowexc = jnp.dot(tri79, rowtot, preferred_element_type=F32,
                     precision=lax.Precision.HIGHEST)
    pos2d = (inner + rowexc).astype(I32)
    basev = sid * CAP
    padj = jnp.where(valid, pos2d + basev, basev + DUMPSLOT)
    pos_ref[...] = padj
    cnt_ref[...] = jnp.full((8, 128), jnp.sum(v2).astype(I32), I32)


@functools.partial(
    pl.kernel,
    out_type=(jax.ShapeDtypeStruct((NW, CAP), I32),
              jax.ShapeDtypeStruct((NW, CAP), I32)), **MESH,
    scratch_types=[pltpu.VMEM_SHARED((16 * CAP,), I32),
                   pltpu.VMEM_SHARED((16 * CAP,), I32),
                   pltpu.VMEM((EPAD,), I32), pltpu.VMEM((EPAD,), I32),
                   pltpu.VMEM((CAP,), I32),
                   pltpu.SemaphoreType.DMA, pltpu.SemaphoreType.DMA])
def _k4c_compact(m0_hbm, m1_hbm, pos_hbm, r_out, c_out,
                 rc_sp, cc_sp, mbuf, posb, zb, sem1, sem2):
    cid, sid, wid = _ids()

    def zfill(i, c):
        zb[pl.ds(i * 16, 16)] = jnp.full((16,), 0, I32)
        return c
    lax.fori_loop(0, CAP // 16, zfill, jnp.int32(0))
    pltpu.sync_copy(zb, rc_sp.at[pl.ds(sid * CAP, CAP)])
    pltpu.sync_copy(zb, cc_sp.at[pl.ds(sid * CAP, CAP)])
    pltpu.sync_copy(pos_hbm.at[wid], posb)
    pltpu.sync_copy(m0_hbm.at[wid], mbuf)
    pltpu.async_copy(mbuf, rc_sp.at[posb], sem1, add=True).wait()
    pltpu.sync_copy(m1_hbm.at[wid], mbuf)
    pltpu.async_copy(mbuf, cc_sp.at[posb], sem2, add=True).wait()
    pltpu.sync_copy(rc_sp.at[pl.ds(sid * CAP, CAP)], zb)
    pltpu.sync_copy(zb, r_out.at[wid])
    pltpu.sync_copy(cc_sp.at[pl.ds(sid * CAP, CAP)], zb)
    pltpu.sync_copy(zb, c_out.at[wid])


@functools.partial(
    pl.kernel, out_type=jax.ShapeDtypeStruct((2, SAMPLE, FEAT), F32), **MESH,
    scratch_types=[pltpu.VMEM_SHARED((TROWS, FEAT), F32),
                   pltpu.VMEM_SHARED((TROWS, FEAT), F32),
                   pltpu.VMEM((STRIPE, FEAT), F32),
                   pltpu.VMEM((CAP,), I32), pltpu.VMEM((CAP,), I32),
                   pltpu.VMEM((16,), I32), pltpu.VMEM((16, FEAT), F32),
                   pltpu.SemaphoreType.DMA, pltpu.SemaphoreType.DMA])
def _k5_agg(sup_hbm, r_hbm, c_hbm, cnt_hbm, z_hbm, out_hbm,
            sup_sp, acc_sp, bnc, rbuf, cbuf, cbr, stage, sg, sa):
    cid, sid, wid = _ids()
    off_acc = jnp.minimum(STRIPE * sid, TROWS - STRIPE)
    off_sup = jnp.minimum(STRIPE * sid, SAMPLE - STRIPE)
    pltpu.sync_copy(z_hbm.at[pl.ds(off_acc, STRIPE)], bnc)
    pltpu.sync_copy(bnc, acc_sp.at[pl.ds(off_acc, STRIPE)])
    pltpu.sync_copy(sup_hbm.at[pl.ds(off_sup, STRIPE)], bnc)
    pltpu.sync_copy(bnc, sup_sp.at[pl.ds(off_sup, STRIPE)])

    @pl.when(sid == 0)
    def _():
        pltpu.sync_copy(z_hbm.at[pl.ds(0, 8)], bnc.at[pl.ds(0, 8)])
        pltpu.sync_copy(bnc.at[pl.ds(0, 8)], sup_sp.at[pl.ds(SAMPLE, 8)])
    plsc.subcore_barrier()

    pltpu.sync_copy(r_hbm.at[wid], rbuf)
    pltpu.sync_copy(c_hbm.at[wid], cbuf)
    pltpu.sync_copy(cnt_hbm.at[pl.ds(wid * 1024, 16)], cbr)
    cnt = cbr[...][0]
    nb = (cnt + 15) // 16

    def blk(b, carry):
        ri = rbuf[pl.ds(b * 16, 16)]
        ci = cbuf[pl.ds(b * 16, 16)]
        lm = (b * 16 + _iota16()) < cnt
        ri = jnp.where(lm, ri, DUMP)
        ci = jnp.where(lm, ci, DUMP)
        pltpu.async_copy(sup_sp.at[ri], stage, sg).wait()
        pltpu.async_copy(stage, acc_sp.at[ci], sa, add=True).wait()
        return carry
    lax.fori_loop(0, nb, blk, jnp.int32(0))
    plsc.subcore_barrier()
    pltpu.sync_copy(acc_sp.at[pl.ds(off_sup, STRIPE)], bnc)
    pltpu.sync_copy(bnc, out_hbm.at[cid, pl.ds(off_sup, STRIPE)])


@functools.partial(
    pl.kernel, out_type=jax.ShapeDtypeStruct((SAMPLE,), F32), **MESH,
    scratch_types=[pltpu.VMEM_SHARED((TROWS,), F32),
                   pltpu.VMEM_SHARED((TROWS,), F32),
                   pltpu.VMEM((STRIPE,), F32),
                   pltpu.VMEM((CAP,), I32), pltpu.VMEM((CAP,), I32),
                   pltpu.VMEM((16,), I32), pltpu.VMEM((16,), F32),
                   pltpu.SemaphoreType.DMA, pltpu.SemaphoreType.DMA])
def _k7_agg1(s3_hbm, r_hbm, c_hbm, cnt_hbm, binit_hbm, out_hbm,
             s3_sp, acc_sp, bnc, rbuf, cbuf, cbr, gb, sg, sa):
    cid, sid, wid = _ids()

    @pl.when(cid == 0)
    def _():
        off_acc = jnp.minimum(STRIPE * sid, TROWS - STRIPE)
        off_s3 = jnp.minimum(STRIPE * sid, SAMPLE - STRIPE)
        pltpu.sync_copy(binit_hbm.at[pl.ds(off_acc, STRIPE)], bnc)
        pltpu.sync_copy(bnc, acc_sp.at[pl.ds(off_acc, STRIPE)])
        pltpu.sync_copy(s3_hbm.at[pl.ds(off_s3, STRIPE)], bnc)
        pltpu.sync_copy(bnc, s3_sp.at[pl.ds(off_s3, STRIPE)])
    plsc.subcore_barrier()

    @pl.when(cid == 0)
    def _():
        def one(t, carry):
            w = t * 16 + sid
            pltpu.sync_copy(r_hbm.at[w], rbuf)
            pltpu.sync_copy(c_hbm.at[w], cbuf)
            pltpu.sync_copy(cnt_hbm.at[pl.ds(w * 1024, 16)], cbr)
            cnt = cbr[...][0]
            nb = (cnt + 15) // 16

            def blk(b, c2):
                ri = rbuf[pl.ds(b * 16, 16)]
                ci = cbuf[pl.ds(b * 16, 16)]
                lm = (b * 16 + _iota16()) < cnt
                ri = jnp.where(lm, ri, DUMP)
                ci = jnp.where(lm, ci, DUMP)
                pltpu.async_copy(s3_sp.at[ri], gb, sg).wait()
                pltpu.async_copy(gb, acc_sp.at[ci], sa, add=True).wait()
                return c2
            lax.fori_loop(0, nb, blk, jnp.int32(0))
            return carry
        lax.fori_loop(0, 2, one, jnp.int32(0))
    plsc.subcore_barrier()

    @pl.when(cid == 0)
    def _():
        off_s3 = jnp.minimum(STRIPE * sid, SAMPLE - STRIPE)
        pltpu.sync_copy(acc_sp.at[pl.ds(off_s3, STRIPE)], bnc)
        pltpu.sync_copy(bnc, out_hbm.at[pl.ds(off_s3, STRIPE)])


def _tc1_body(xs_ref, w_ref, o_ref):
    o_ref[...] = jnp.dot(xs_ref[...], w_ref[...], preferred_element_type=F32)


def _bn_relu(p_ref, b_ref, g_ref, bb_ref):
    h = p_ref[0] + p_ref[1] + b_ref[...]
    h = h / jnp.sqrt(jnp.float32(1.0 + 1e-5)) * g_ref[...] + bb_ref[...]
    return jnp.maximum(h, 0.0)


def _tc2_body(p_ref, b_ref, g_ref, bb_ref, w_ref, o_ref):
    h = _bn_relu(p_ref, b_ref, g_ref, bb_ref)
    o_ref[...] = jnp.dot(h, w_ref[...], preferred_element_type=F32)


def _tc3_body(p_ref, b_ref, g_ref, bb_ref, w3_ref, o_ref):
    h = _bn_relu(p_ref, b_ref, g_ref, bb_ref)
    s = jnp.sum(h * w3_ref[...], axis=1)
    o_ref[...] = s.reshape(8, 128)


def kernel(x, edge_index, W1, b1, W2, b2, W3, b3, bn1_g, bn1_b, bn2_g, bn2_b):
    row = edge_index[0]
    col = edge_index[1]

    degp = _k1_deg(row).reshape(2, N_NODES)
    deg = degp[0] + degp[1]
    prob = deg / jnp.sum(deg)
    p_cuml = jnp.cumsum(prob)
    om_u = 1.0 - jax.random.uniform(jax.random.key(42), (SAMPLE,), F32)
    rvals = p_cuml[-1] * om_u
    sampled = _k2_search(p_cuml, rvals)
    mpart = _k3_idxmap(sampled).reshape(2, N_NODES)
    idx_map = jnp.maximum(mpart[0], mpart[1])

    m0, m1, x_s = _k4a_map(row, col, idx_map, sampled, x)
    m0r = m0.reshape(NW * 80, 128)
    m1r = m1.reshape(NW * 80, 128)
    posadj, cnts = pl.pallas_call(
        _k4b_body,
        grid=(NW,),
        in_specs=[pl.BlockSpec((80, 128), lambda i: (i, 0)),
                  pl.BlockSpec((80, 128), lambda i: (i, 0))],
        out_specs=[pl.BlockSpec((80, 128), lambda i: (i, 0)),
                   pl.BlockSpec((8, 128), lambda i: (i, 0))],
        out_shape=[jax.ShapeDtypeStruct((NW * 80, 128), I32),
                   jax.ShapeDtypeStruct((NW * 8, 128), I32)],
    )(m0r, m1r)
    posadj = posadj.reshape(NW, EPAD)
    cnts = cnts.reshape(NW * 8 * 128)
    rlist, clist = _k4c_compact(m0, m1, posadj)

    zeros2d = jnp.zeros((TROWS, FEAT), F32)

    sup1 = pl.pallas_call(
        _tc1_body, grid=(8,),
        in_specs=[pl.BlockSpec((512, FEAT), lambda i: (i, 0)),
                  pl.BlockSpec((FEAT, FEAT), lambda i: (0, 0))],
        out_specs=pl.BlockSpec((512, FEAT), lambda i: (i, 0)),
        out_shape=jax.ShapeDtypeStruct((SAMPLE, FEAT), F32))(x_s, W1)
    agg1 = _k5_agg(sup1, rlist, clist, cnts, zeros2d)

    sup2 = pl.pallas_call(
        _tc2_body, grid=(8,),
        in_specs=[pl.BlockSpec((2, 512, FEAT), lambda i: (0, i, 0)),
                  pl.BlockSpec((1, FEAT), lambda i: (0, 0)),
                  pl.BlockSpec((1, FEAT), lambda i: (0, 0)),
                  pl.BlockSpec((1, FEAT), lambda i: (0, 0)),
                  pl.BlockSpec((FEAT, FEAT), lambda i: (0, 0))],
        out_specs=pl.BlockSpec((512, FEAT), lambda i: (i, 0)),
        out_shape=jax.ShapeDtypeStruct((SAMPLE, FEAT), F32))(
            agg1, b1.reshape(1, FEAT), bn1_g.reshape(1, FEAT),
            bn1_b.reshape(1, FEAT), W2)
    agg2 = _k5_agg(sup2, rlist, clist, cnts, zeros2d)

    s3 = pl.pallas_call(
        _tc3_body, grid=(4,),
        in_specs=[pl.BlockSpec((2, 1024, FEAT), lambda i: (0, i, 0)),
                  pl.BlockSpec((1, FEAT), lambda i: (0, 0)),
                  pl.BlockSpec((1, FEAT), lambda i: (0, 0)),
                  pl.BlockSpec((1, FEAT), lambda i: (0, 0)),
                  pl.BlockSpec((1, FEAT), lambda i: (0, 0))],
        out_specs=pl.BlockSpec((8, 128), lambda i: (i, 0)),
        out_shape=jax.ShapeDtypeStruct((32, 128), F32))(
            agg2, b2.reshape(1, FEAT), bn2_g.reshape(1, FEAT),
            bn2_b.reshape(1, FEAT), W3.reshape(1, FEAT))
    s3 = s3.reshape(SAMPLE)
    binit = jnp.broadcast_to(b3, (TROWS,))
    out3 = _k7_agg1(s3, rlist, clist, cnts, binit)
    return (out3.reshape(SAMPLE, 1), sampled)
